# baseline (speedup 1.0000x reference)
import functools

import jax
import jax.numpy as jnp
from jax import lax
from jax.experimental import pallas as pl
from jax.experimental.pallas import tpu as pltpu
from jax.experimental.pallas import tpu_sc as plsc

EMB_D = 8


def _make_sc_gather(B, D, n_workers, num_cores):
  b_per_w = B // n_workers
  mesh = plsc.VectorSubcoreMesh(core_axis_name="c", subcore_axis_name="s")

  @functools.partial(
      pl.kernel,
      mesh=mesh,
      compiler_params=pltpu.CompilerParams(use_tc_tiling_on_sc=False),
      out_type=(
          jax.ShapeDtypeStruct((B, D), jnp.float32),
          jax.ShapeDtypeStruct((B, D), jnp.float32),
      ),
      scratch_types=[
          pltpu.VMEM((b_per_w,), jnp.int32),
          pltpu.VMEM((b_per_w, D), jnp.float32),
          pltpu.VMEM((b_per_w,), jnp.int32),
          pltpu.VMEM((b_per_w, D), jnp.float32),
          pltpu.SemaphoreType.DMA,
          pltpu.SemaphoreType.DMA,
      ],
  )
  def sc_gather(u_hbm, uid_hbm, t_hbm, tid_hbm, out_u, out_t,
                uidx_v, urows_v, tidx_v, trows_v, sem_u, sem_t):
    wid = lax.axis_index("s") * num_cores + lax.axis_index("c")
    base = wid * b_per_w
    pltpu.sync_copy(uid_hbm.at[pl.ds(base, b_per_w)], uidx_v)
    pltpu.sync_copy(tid_hbm.at[pl.ds(base, b_per_w)], tidx_v)
    cp_u = pltpu.async_copy(u_hbm.at[uidx_v], urows_v, sem_u)
    cp_t = pltpu.async_copy(t_hbm.at[tidx_v], trows_v, sem_t)
    cp_u.wait()
    cp_t.wait()
    pltpu.sync_copy(urows_v, out_u.at[pl.ds(base, b_per_w)])
    pltpu.sync_copy(trows_v, out_t.at[pl.ds(base, b_per_w)])

  return sc_gather


def _mlp_body(ue_ref, te_ref, uc_ref, ac_ref,
              w1_ref, b1_ref, w2_ref, b2_ref, w3_ref, b3_ref, w4_ref, b4_ref,
              out_ref):
  dot = functools.partial(jnp.dot, precision=lax.Precision.HIGHEST,
                          preferred_element_type=jnp.float32)
  h = dot(ue_ref[...], w1_ref[0:EMB_D, :])
  h += dot(te_ref[...], w1_ref[EMB_D:2 * EMB_D, :])
  h += dot(uc_ref[...], w1_ref[2 * EMB_D:2 * EMB_D + 64, :])
  h += dot(ac_ref[...], w1_ref[2 * EMB_D + 64:, :])
  h = jnp.maximum(h + b1_ref[...], 0.0)
  h = jnp.maximum(dot(h, w2_ref[...]) + b2_ref[...], 0.0)
  h = jnp.maximum(dot(h, w3_ref[...]) + b3_ref[...], 0.0)
  out = dot(h, w4_ref[...]) + b4_ref[...]
  out_ref[...] = out[:, 0]


def _mlp_call(ue, te, uc, ac, W1, b1, W2, b2, W3, b3, W4, b4, blk):
  B = ue.shape[0]
  grid = (B // blk,)

  def row_spec(d):
    return pl.BlockSpec((blk, d), lambda i: (i, 0))

  def full_spec(shape):
    nd = len(shape)
    return pl.BlockSpec(shape, lambda i: (0,) * nd)

  return pl.pallas_call(
      _mlp_body,
      grid=grid,
      in_specs=[
          row_spec(EMB_D), row_spec(EMB_D), row_spec(64), row_spec(64),
          full_spec(W1.shape), full_spec(b1.shape),
          full_spec(W2.shape), full_spec(b2.shape),
          full_spec(W3.shape), full_spec(b3.shape),
          full_spec(W4.shape), full_spec(b4.shape),
      ],
      out_specs=pl.BlockSpec((blk,), lambda i: (i,)),
      out_shape=jax.ShapeDtypeStruct((B,), jnp.float32),
  )(ue, te, uc, ac, W1, b1, W2, b2, W3, b3, W4, b4)


@jax.jit
def kernel(user_id, artist_id, user_country, artist_country, U, T,
           W1, b1, W2, b2, W3, b3, W4, b4):
  B = user_id.shape[0]
  mesh = plsc.VectorSubcoreMesh(core_axis_name="c", subcore_axis_name="s")
  n_workers = mesh.num_cores * mesh.num_subcores
  gather = _make_sc_gather(B, EMB_D, n_workers, mesh.num_cores)
  ue, te = gather(U, user_id, T, artist_id)
  return _mlp_call(ue, te, user_country, artist_country,
                   W1, b1, W2, b2, W3, b3, W4, b4, blk=2048)
